# baseline (speedup 1.0000x reference)
import functools

import jax
import jax.numpy as jnp
from jax import lax
from jax.experimental import pallas as pl
from jax.experimental.pallas import tpu as pltpu
from jax.experimental.pallas import tpu_sc as plsc

N = 10000
NP = 10240
D_IN = 128
D_HID = 16
D_OUT = 128
E_REAL = 320000
CHUNK = 128
NCHUNK = 80
PER_TILE = E_REAL // 32
FULL_CHUNKS = PER_TILE // CHUNK
REM = PER_TILE - FULL_CHUNKS * CHUNK
ROWS_PER_TILE = NP // 16

_mesh = plsc.VectorSubcoreMesh(core_axis_name="c", subcore_axis_name="s")
_params = pltpu.CompilerParams(use_tc_tiling_on_sc=False)


def _newton_rsqrt(x):
  i = lax.bitcast_convert_type(x, jnp.int32)
  i = jnp.full((16,), 0x5F3759DF, jnp.int32) - lax.shift_right_arithmetic(
      i, jnp.full((16,), 1, jnp.int32))
  y = lax.bitcast_convert_type(i, jnp.float32)
  half = x * (-0.5)
  for _ in range(3):
    y = y * (half * y * y + 1.5)
  return y


def _load_edges(e_hbm, row, buf, base, sem, pad_val):
  handles = [
      pltpu.async_copy(e_hbm.at[row, pl.ds(base + j * CHUNK, CHUNK)],
                       buf.at[j], sem)
      for j in range(FULL_CHUNKS)
  ]
  handles.append(
      pltpu.async_copy(e_hbm.at[row, pl.ds(base + FULL_CHUNKS * CHUNK, REM)],
                       buf.at[FULL_CHUNKS, pl.ds(0, REM)], sem))
  pad = jnp.full((16,), pad_val, jnp.int32)
  for k in range(REM // 16, CHUNK // 16):
    buf[FULL_CHUNKS, pl.ds(16 * k, 16)] = pad
  for k in range(CHUNK // 16):
    buf[NCHUNK - 1, pl.ds(16 * k, 16)] = pad
  return handles


def _edge_loop(do_gather, dummy_hbm, featsh, acc, srcv, dstv, rowsv,
               gsem, ssem_a, ssem_b):
  ssems = (ssem_a, ssem_b)
  NG = NCHUNK // 8

  def _drain(b, sem):
    pltpu.make_async_copy(dummy_hbm.at[pl.ds(0, CHUNK)], rowsv.at[b],
                          sem).wait()

  if do_gather:
    for b in range(8):
      pltpu.async_copy(featsh.at[srcv.at[b]], rowsv.at[b], gsem)

  def pair(g2, carry):
    for p in range(2):
      g = g2 * 2 + p
      po = 8 * p
      qo = 8 * (1 - p)
      if do_gather:
        for b in range(8):
          _drain(po + b, gsem)

      @pl.when(g > 0)
      def _():
        for b in range(8):
          _drain(qo + b, ssems[1 - p])

      if do_gather:
        @pl.when(g + 1 < NG)
        def _():
          for b in range(8):
            pltpu.async_copy(featsh.at[srcv.at[(g + 1) * 8 + b]],
                             rowsv.at[qo + b], gsem)

      for b in range(8):
        pltpu.async_copy(rowsv.at[po + b], acc.at[dstv.at[g * 8 + b]],
                         ssems[p], add=True)
    return carry

  lax.fori_loop(0, NG // 2, pair, 0)
  last = (NG - 1) % 2
  for b in range(8):
    _drain(8 * last + b, ssems[last])


def _tile_ids():
  c = lax.axis_index("c")
  s = lax.axis_index("s")
  return c, s, s * 2 + c, s * ROWS_PER_TILE


def _drain_out(acc, tmpv, out_hbm, c, r0):
  plsc.subcore_barrier()
  pltpu.sync_copy(acc.at[pl.ds(r0, ROWS_PER_TILE)], tmpv)
  pltpu.sync_copy(tmpv, out_hbm.at[c, pl.ds(r0, ROWS_PER_TILE), :])


def _deg_body(ones_hbm, e_hbm, out_hbm, dstv, rowsv, tmpv, acc, featsh,
              gsem, ssem_a, ssem_b):
  c, s, wid, r0 = _tile_ids()
  handles = _load_edges(e_hbm, 1, dstv, wid * PER_TILE, gsem, NP - 1)
  for b in range(16):
    pltpu.sync_copy(ones_hbm.at[pl.ds(0, CHUNK)], rowsv.at[b])
  pltpu.sync_copy(ones_hbm.at[pl.ds(r0, ROWS_PER_TILE)], tmpv)
  pltpu.sync_copy(tmpv, acc.at[pl.ds(r0, ROWS_PER_TILE)])
  for h in handles:
    h.wait()
  plsc.subcore_barrier()
  _edge_loop(False, ones_hbm, featsh, acc, None, dstv, rowsv,
             gsem, ssem_a, ssem_b)
  _drain_out(acc, tmpv, out_hbm, c, r0)


_sc_deg = functools.partial(
    pl.kernel,
    out_type=jax.ShapeDtypeStruct((2, NP, D_HID), jnp.float32),
    mesh=_mesh,
    scratch_types=[
        pltpu.VMEM((NCHUNK, CHUNK), jnp.int32),
        pltpu.VMEM((16, CHUNK, D_HID), jnp.float32),
        pltpu.VMEM((ROWS_PER_TILE, D_HID), jnp.float32),
        pltpu.VMEM_SHARED((NP, D_HID), jnp.float32),
        pltpu.VMEM_SHARED((NP, D_HID), jnp.float32),
        pltpu.SemaphoreType.DMA,
        pltpu.SemaphoreType.DMA,
        pltpu.SemaphoreType.DMA,
    ],
    compiler_params=_params,
)(_deg_body)


def _prop1_body(p1_hbm, degp_hbm, e_hbm, out_hbm,
                srcv, dstv, rowsv, av, bv, acc, featsh,
                gsem, ssem_a, ssem_b):
  c, s, wid, r0 = _tile_ids()
  handles = _load_edges(e_hbm, 1, dstv, wid * PER_TILE, gsem, NP - 1)
  handles += _load_edges(e_hbm, 0, srcv, wid * PER_TILE, gsem, 0)
  pltpu.sync_copy(degp_hbm.at[0, pl.ds(r0, ROWS_PER_TILE), :], av)
  pltpu.sync_copy(degp_hbm.at[1, pl.ds(r0, ROWS_PER_TILE), :], bv)

  def dinv_loop(j, carry):
    av[j] = _newton_rsqrt(av[j] + bv[j] - 1.0)
    return carry

  lax.fori_loop(0, ROWS_PER_TILE, dinv_loop, 0)
  pltpu.sync_copy(p1_hbm.at[pl.ds(r0, ROWS_PER_TILE)], bv)

  def scale(j, carry):
    bv[j] = bv[j] * av[j]
    return carry

  lax.fori_loop(0, ROWS_PER_TILE, scale, 0)
  pltpu.sync_copy(bv, acc.at[pl.ds(r0, ROWS_PER_TILE)])
  pltpu.sync_copy(bv, featsh.at[pl.ds(r0, ROWS_PER_TILE)])
  for h in handles:
    h.wait()
  plsc.subcore_barrier()
  _edge_loop(True, p1_hbm, featsh, acc, srcv, dstv, rowsv,
             gsem, ssem_a, ssem_b)
  _drain_out(acc, av, out_hbm, c, r0)


_sc_prop1 = functools.partial(
    pl.kernel,
    out_type=jax.ShapeDtypeStruct((2, NP, D_HID), jnp.float32),
    mesh=_mesh,
    scratch_types=[
        pltpu.VMEM((NCHUNK, CHUNK), jnp.int32),
        pltpu.VMEM((NCHUNK, CHUNK), jnp.int32),
        pltpu.VMEM((16, CHUNK, D_HID), jnp.float32),
        pltpu.VMEM((ROWS_PER_TILE, D_HID), jnp.float32),
        pltpu.VMEM((ROWS_PER_TILE, D_HID), jnp.float32),
        pltpu.VMEM_SHARED((NP, D_HID), jnp.float32),
        pltpu.VMEM_SHARED((NP, D_HID), jnp.float32),
        pltpu.SemaphoreType.DMA,
        pltpu.SemaphoreType.DMA,
        pltpu.SemaphoreType.DMA,
    ],
    compiler_params=_params,
)(_prop1_body)


def _prop2_body(m1_hbm, p1_hbm, degp_hbm, b1_hbm, e_hbm, m2_hbm, hp_hbm,
                srcv, dstv, rowsv, av, bv, cv, dv, b1v,
                acc, featsh, gsem, ssem_a, ssem_b):
  c, s, wid, r0 = _tile_ids()
  handles = _load_edges(e_hbm, 1, dstv, wid * PER_TILE, gsem, NP - 1)
  handles += _load_edges(e_hbm, 0, srcv, wid * PER_TILE, gsem, 0)
  pltpu.sync_copy(degp_hbm.at[0, pl.ds(r0, ROWS_PER_TILE), :], av)
  pltpu.sync_copy(degp_hbm.at[1, pl.ds(r0, ROWS_PER_TILE), :], bv)

  def dinv_loop(j, carry):
    av[j] = _newton_rsqrt(av[j] + bv[j] - 1.0)
    return carry

  lax.fori_loop(0, ROWS_PER_TILE, dinv_loop, 0)
  pltpu.sync_copy(p1_hbm.at[pl.ds(r0, ROWS_PER_TILE)], bv)
  pltpu.sync_copy(m1_hbm.at[0, pl.ds(r0, ROWS_PER_TILE), :], cv)
  pltpu.sync_copy(m1_hbm.at[1, pl.ds(r0, ROWS_PER_TILE), :], dv)
  pltpu.sync_copy(b1_hbm, b1v)
  b1row = b1v[0]

  def mid(j, carry):
    dinv = av[j]
    m = cv[j] + dv[j] - bv[j] * dinv
    h = jnp.maximum(m * dinv + b1row, 0.0)
    dv[j] = h * dinv
    return carry

  lax.fori_loop(0, ROWS_PER_TILE, mid, 0)
  pltpu.sync_copy(dv, acc.at[pl.ds(r0, ROWS_PER_TILE)])
  pltpu.sync_copy(dv, featsh.at[pl.ds(r0, ROWS_PER_TILE)])
  pltpu.sync_copy(dv, hp_hbm.at[pl.ds(r0, ROWS_PER_TILE)])
  for h in handles:
    h.wait()
  plsc.subcore_barrier()
  _edge_loop(True, p1_hbm, featsh, acc, srcv, dstv, rowsv,
             gsem, ssem_a, ssem_b)
  _drain_out(acc, dv, m2_hbm, c, r0)


_sc_prop2 = functools.partial(
    pl.kernel,
    out_type=(jax.ShapeDtypeStruct((2, NP, D_HID), jnp.float32),
              jax.ShapeDtypeStruct((NP, D_HID), jnp.float32)),
    mesh=_mesh,
    scratch_types=[
        pltpu.VMEM((NCHUNK, CHUNK), jnp.int32),
        pltpu.VMEM((NCHUNK, CHUNK), jnp.int32),
        pltpu.VMEM((16, CHUNK, D_HID), jnp.float32),
        pltpu.VMEM((ROWS_PER_TILE, D_HID), jnp.float32),
        pltpu.VMEM((ROWS_PER_TILE, D_HID), jnp.float32),
        pltpu.VMEM((ROWS_PER_TILE, D_HID), jnp.float32),
        pltpu.VMEM((ROWS_PER_TILE, D_HID), jnp.float32),
        pltpu.VMEM((1, D_HID), jnp.float32),
        pltpu.VMEM_SHARED((NP, D_HID), jnp.float32),
        pltpu.VMEM_SHARED((NP, D_HID), jnp.float32),
        pltpu.SemaphoreType.DMA,
        pltpu.SemaphoreType.DMA,
        pltpu.SemaphoreType.DMA,
    ],
    compiler_params=_params,
)(_prop2_body)


def _tc_matmul(x_ref, w1_ref, out_ref):
  out_ref[...] = jnp.dot(x_ref[...], w1_ref[...],
                         preferred_element_type=jnp.float32)


def _tc_final(m_ref, hp_ref, deg_ref, w2_ref, b2_ref, out_ref):
  dinv = lax.rsqrt(deg_ref[0] + deg_ref[1] - 1.0)
  m = (m_ref[0] + m_ref[1] - hp_ref[...]) * dinv
  out_ref[...] = jnp.dot(m, w2_ref[...],
                         preferred_element_type=jnp.float32) + b2_ref[...]


def kernel(V, E, X, W1, b1, W2, b2):
  del V
  f32 = jnp.float32
  ones = jnp.ones((NP, D_HID), f32)
  Xp = jnp.concatenate([X, jnp.zeros((NP - N, D_IN), f32)])

  degp = _sc_deg(ones, E)
  p1 = pl.pallas_call(
      _tc_matmul,
      out_shape=jax.ShapeDtypeStruct((NP, D_HID), f32),
  )(Xp, W1)

  m1 = _sc_prop1(p1, degp, E)
  m2, hp = _sc_prop2(m1, p1, degp, b1.reshape(1, D_HID), E)

  out = pl.pallas_call(
      _tc_final,
      out_shape=jax.ShapeDtypeStruct((NP, D_OUT), f32),
  )(m2, hp, degp, W2, b2.reshape(1, D_OUT))
  return out[:N]


# speedup vs baseline: 1.1850x; 1.1850x over previous
import functools

import jax
import jax.numpy as jnp
from jax import lax
from jax.experimental import pallas as pl
from jax.experimental.pallas import tpu as pltpu
from jax.experimental.pallas import tpu_sc as plsc

N = 10000
NP = 10240
D_IN = 128
D_HID = 16
D_OUT = 128
E_REAL = 320000
CHUNK = 128
NCHUNK = 80
PER_TILE = E_REAL // 32
FULL_CHUNKS = PER_TILE // CHUNK
REM = PER_TILE - FULL_CHUNKS * CHUNK
ROWS_PER_TILE = NP // 16

_mesh = plsc.VectorSubcoreMesh(core_axis_name="c", subcore_axis_name="s")
_params = pltpu.CompilerParams(use_tc_tiling_on_sc=False)


def _newton_rsqrt(x):
  i = lax.bitcast_convert_type(x, jnp.int32)
  i = jnp.full((16,), 0x5F3759DF, jnp.int32) - lax.shift_right_arithmetic(
      i, jnp.full((16,), 1, jnp.int32))
  y = lax.bitcast_convert_type(i, jnp.float32)
  half = x * (-0.5)
  for _ in range(3):
    y = y * (half * y * y + 1.5)
  return y


def _load_edges(e_hbm, row, buf, base, sem, pad_val):
  handles = [
      pltpu.async_copy(e_hbm.at[row, pl.ds(base + j * CHUNK, CHUNK)],
                       buf.at[j], sem)
      for j in range(FULL_CHUNKS)
  ]
  handles.append(
      pltpu.async_copy(e_hbm.at[row, pl.ds(base + FULL_CHUNKS * CHUNK, REM)],
                       buf.at[FULL_CHUNKS, pl.ds(0, REM)], sem))
  pad = jnp.full((16,), pad_val, jnp.int32)
  for k in range(REM // 16, CHUNK // 16):
    buf[FULL_CHUNKS, pl.ds(16 * k, 16)] = pad
  for k in range(CHUNK // 16):
    buf[NCHUNK - 1, pl.ds(16 * k, 16)] = pad
  return handles


def _edge_loop(do_gather, dummy_hbm, featsh, acc, srcv, dstv, rowsv,
               gsem, ssem_a, ssem_b):
  ssems = (ssem_a, ssem_b)
  NG = NCHUNK // 8

  def _drain(b, sem):
    pltpu.make_async_copy(dummy_hbm.at[pl.ds(0, CHUNK)], rowsv.at[b],
                          sem).wait()

  if do_gather:
    for b in range(8):
      pltpu.async_copy(featsh.at[srcv.at[b]], rowsv.at[b], gsem)

  def pair(g2, carry):
    for p in range(2):
      g = g2 * 2 + p
      po = 8 * p
      qo = 8 * (1 - p)
      if do_gather:
        for b in range(8):
          _drain(po + b, gsem)

      @pl.when(g > 0)
      def _():
        for b in range(8):
          _drain(qo + b, ssems[1 - p])

      if do_gather:
        @pl.when(g + 1 < NG)
        def _():
          for b in range(8):
            pltpu.async_copy(featsh.at[srcv.at[(g + 1) * 8 + b]],
                             rowsv.at[qo + b], gsem)

      for b in range(8):
        pltpu.async_copy(rowsv.at[po + b], acc.at[dstv.at[g * 8 + b]],
                         ssems[p], add=True)
    return carry

  lax.fori_loop(0, NG // 2, pair, 0)
  last = (NG - 1) % 2
  for b in range(8):
    _drain(8 * last + b, ssems[last])


def _tile_ids():
  c = lax.axis_index("c")
  s = lax.axis_index("s")
  return c, s, s * 2 + c, s * ROWS_PER_TILE


def _drain_out(acc, tmpv, out_hbm, c, r0):
  plsc.subcore_barrier()
  pltpu.sync_copy(acc.at[pl.ds(r0, ROWS_PER_TILE)], tmpv)
  pltpu.sync_copy(tmpv, out_hbm.at[c, pl.ds(r0, ROWS_PER_TILE), :])


def _deg_body(ones_hbm, e_hbm, out_hbm, dstv, rowsv, tmpv, acc, featsh,
              gsem, ssem_a, ssem_b):
  c, s, wid, r0 = _tile_ids()
  handles = _load_edges(e_hbm, 1, dstv, wid * PER_TILE, gsem, NP - 1)
  handles += [
      pltpu.async_copy(ones_hbm.at[pl.ds(0, CHUNK)], rowsv.at[b], ssem_a)
      for b in range(16)
  ]
  pltpu.sync_copy(ones_hbm.at[pl.ds(r0, ROWS_PER_TILE)], tmpv)
  pltpu.sync_copy(tmpv, acc.at[pl.ds(r0, ROWS_PER_TILE)])
  for h in handles:
    h.wait()
  plsc.subcore_barrier()
  _edge_loop(False, ones_hbm, featsh, acc, None, dstv, rowsv,
             gsem, ssem_a, ssem_b)
  _drain_out(acc, tmpv, out_hbm, c, r0)


_sc_deg = functools.partial(
    pl.kernel,
    out_type=jax.ShapeDtypeStruct((2, NP, D_HID), jnp.float32),
    mesh=_mesh,
    scratch_types=[
        pltpu.VMEM((NCHUNK, CHUNK), jnp.int32),
        pltpu.VMEM((16, CHUNK, D_HID), jnp.float32),
        pltpu.VMEM((ROWS_PER_TILE, D_HID), jnp.float32),
        pltpu.VMEM_SHARED((NP, D_HID), jnp.float32),
        pltpu.VMEM_SHARED((NP, D_HID), jnp.float32),
        pltpu.SemaphoreType.DMA,
        pltpu.SemaphoreType.DMA,
        pltpu.SemaphoreType.DMA,
    ],
    compiler_params=_params,
)(_deg_body)


def _prop1_body(p1_hbm, degp_hbm, e_hbm, out_hbm,
                srcv, dstv, rowsv, av, bv, acc, featsh,
                gsem, ssem_a, ssem_b):
  c, s, wid, r0 = _tile_ids()
  handles = _load_edges(e_hbm, 1, dstv, wid * PER_TILE, gsem, NP - 1)
  handles += _load_edges(e_hbm, 0, srcv, wid * PER_TILE, gsem, 0)
  pltpu.sync_copy(degp_hbm.at[0, pl.ds(r0, ROWS_PER_TILE), :], av)
  pltpu.sync_copy(degp_hbm.at[1, pl.ds(r0, ROWS_PER_TILE), :], bv)

  def dinv_loop(j4, carry):
    for k in range(4):
      j = j4 * 4 + k
      av[j] = _newton_rsqrt(av[j] + bv[j] - 1.0)
    return carry

  lax.fori_loop(0, ROWS_PER_TILE // 4, dinv_loop, 0)
  pltpu.sync_copy(p1_hbm.at[pl.ds(r0, ROWS_PER_TILE)], bv)

  def scale(j4, carry):
    for k in range(4):
      j = j4 * 4 + k
      bv[j] = bv[j] * av[j]
    return carry

  lax.fori_loop(0, ROWS_PER_TILE // 4, scale, 0)
  pltpu.sync_copy(bv, acc.at[pl.ds(r0, ROWS_PER_TILE)])
  pltpu.sync_copy(bv, featsh.at[pl.ds(r0, ROWS_PER_TILE)])
  for h in handles:
    h.wait()
  plsc.subcore_barrier()
  _edge_loop(True, p1_hbm, featsh, acc, srcv, dstv, rowsv,
             gsem, ssem_a, ssem_b)
  _drain_out(acc, av, out_hbm, c, r0)


_sc_prop1 = functools.partial(
    pl.kernel,
    out_type=jax.ShapeDtypeStruct((2, NP, D_HID), jnp.float32),
    mesh=_mesh,
    scratch_types=[
        pltpu.VMEM((NCHUNK, CHUNK), jnp.int32),
        pltpu.VMEM((NCHUNK, CHUNK), jnp.int32),
        pltpu.VMEM((16, CHUNK, D_HID), jnp.float32),
        pltpu.VMEM((ROWS_PER_TILE, D_HID), jnp.float32),
        pltpu.VMEM((ROWS_PER_TILE, D_HID), jnp.float32),
        pltpu.VMEM_SHARED((NP, D_HID), jnp.float32),
        pltpu.VMEM_SHARED((NP, D_HID), jnp.float32),
        pltpu.SemaphoreType.DMA,
        pltpu.SemaphoreType.DMA,
        pltpu.SemaphoreType.DMA,
    ],
    compiler_params=_params,
)(_prop1_body)


def _prop2_body(m1_hbm, p1_hbm, degp_hbm, b1_hbm, e_hbm, m2_hbm, hp_hbm,
                srcv, dstv, rowsv, av, bv, cv, dv, b1v,
                acc, featsh, gsem, ssem_a, ssem_b):
  c, s, wid, r0 = _tile_ids()
  handles = _load_edges(e_hbm, 1, dstv, wid * PER_TILE, gsem, NP - 1)
  handles += _load_edges(e_hbm, 0, srcv, wid * PER_TILE, gsem, 0)
  pltpu.sync_copy(degp_hbm.at[0, pl.ds(r0, ROWS_PER_TILE), :], av)
  pltpu.sync_copy(degp_hbm.at[1, pl.ds(r0, ROWS_PER_TILE), :], bv)

  def dinv_loop(j4, carry):
    for k in range(4):
      j = j4 * 4 + k
      av[j] = _newton_rsqrt(av[j] + bv[j] - 1.0)
    return carry

  lax.fori_loop(0, ROWS_PER_TILE // 4, dinv_loop, 0)
  pltpu.sync_copy(p1_hbm.at[pl.ds(r0, ROWS_PER_TILE)], bv)
  pltpu.sync_copy(m1_hbm.at[0, pl.ds(r0, ROWS_PER_TILE), :], cv)
  pltpu.sync_copy(m1_hbm.at[1, pl.ds(r0, ROWS_PER_TILE), :], dv)
  pltpu.sync_copy(b1_hbm, b1v)
  b1row = b1v[0]

  def mid(j4, carry):
    for k in range(4):
      j = j4 * 4 + k
      dinv = av[j]
      m = cv[j] + dv[j] - bv[j] * dinv
      h = jnp.maximum(m * dinv + b1row, 0.0)
      dv[j] = h * dinv
    return carry

  lax.fori_loop(0, ROWS_PER_TILE // 4, mid, 0)
  pltpu.sync_copy(dv, acc.at[pl.ds(r0, ROWS_PER_TILE)])
  pltpu.sync_copy(dv, featsh.at[pl.ds(r0, ROWS_PER_TILE)])
  pltpu.sync_copy(dv, hp_hbm.at[pl.ds(r0, ROWS_PER_TILE)])
  for h in handles:
    h.wait()
  plsc.subcore_barrier()
  _edge_loop(True, p1_hbm, featsh, acc, srcv, dstv, rowsv,
             gsem, ssem_a, ssem_b)
  _drain_out(acc, dv, m2_hbm, c, r0)


_sc_prop2 = functools.partial(
    pl.kernel,
    out_type=(jax.ShapeDtypeStruct((2, NP, D_HID), jnp.float32),
              jax.ShapeDtypeStruct((NP, D_HID), jnp.float32)),
    mesh=_mesh,
    scratch_types=[
        pltpu.VMEM((NCHUNK, CHUNK), jnp.int32),
        pltpu.VMEM((NCHUNK, CHUNK), jnp.int32),
        pltpu.VMEM((16, CHUNK, D_HID), jnp.float32),
        pltpu.VMEM((ROWS_PER_TILE, D_HID), jnp.float32),
        pltpu.VMEM((ROWS_PER_TILE, D_HID), jnp.float32),
        pltpu.VMEM((ROWS_PER_TILE, D_HID), jnp.float32),
        pltpu.VMEM((ROWS_PER_TILE, D_HID), jnp.float32),
        pltpu.VMEM((1, D_HID), jnp.float32),
        pltpu.VMEM_SHARED((NP, D_HID), jnp.float32),
        pltpu.VMEM_SHARED((NP, D_HID), jnp.float32),
        pltpu.SemaphoreType.DMA,
        pltpu.SemaphoreType.DMA,
        pltpu.SemaphoreType.DMA,
    ],
    compiler_params=_params,
)(_prop2_body)


def _tc_matmul(x_ref, w1_ref, out_ref):
  out_ref[...] = jnp.dot(x_ref[...], w1_ref[...],
                         preferred_element_type=jnp.float32)


def _tc_final(m_ref, hp_ref, deg_ref, w2_ref, b2_ref, out_ref):
  dinv = lax.rsqrt(deg_ref[0] + deg_ref[1] - 1.0)
  m = ((m_ref[0] + m_ref[1] - hp_ref[...]) * dinv)[:N]
  out_ref[...] = jnp.dot(m, w2_ref[...],
                         preferred_element_type=jnp.float32) + b2_ref[...]


def kernel(V, E, X, W1, b1, W2, b2):
  del V
  f32 = jnp.float32
  ones = jnp.ones((NP, D_HID), f32)
  Xp = jnp.concatenate([X, jnp.zeros((NP - N, D_IN), f32)])

  degp = _sc_deg(ones, E)
  p1 = pl.pallas_call(
      _tc_matmul,
      out_shape=jax.ShapeDtypeStruct((NP, D_HID), f32),
  )(Xp, W1)

  m1 = _sc_prop1(p1, degp, E)
  m2, hp = _sc_prop2(m1, p1, degp, b1.reshape(1, D_HID), E)

  out = pl.pallas_call(
      _tc_final,
      out_shape=jax.ShapeDtypeStruct((N, D_OUT), f32),
  )(m2, hp, degp, W2, b2.reshape(1, D_OUT))
  return out


# speedup vs baseline: 1.2912x; 1.0896x over previous
import functools

import jax
import jax.numpy as jnp
from jax import lax
from jax.experimental import pallas as pl
from jax.experimental.pallas import tpu as pltpu
from jax.experimental.pallas import tpu_sc as plsc

N = 10000
NP = 10240
D_IN = 128
D_HID = 16
D_OUT = 128
E_REAL = 320000
CHUNK = 128
NCHUNK = 80
PER_TILE = E_REAL // 32
FULL_CHUNKS = PER_TILE // CHUNK
REM = PER_TILE - FULL_CHUNKS * CHUNK
ROWS_PER_TILE = NP // 16

_mesh = plsc.VectorSubcoreMesh(core_axis_name="c", subcore_axis_name="s")
_params = pltpu.CompilerParams(use_tc_tiling_on_sc=False)


def _newton_rsqrt(x):
  i = lax.bitcast_convert_type(x, jnp.int32)
  i = jnp.full((16,), 0x5F3759DF, jnp.int32) - lax.shift_right_arithmetic(
      i, jnp.full((16,), 1, jnp.int32))
  y = lax.bitcast_convert_type(i, jnp.float32)
  half = x * (-0.5)
  for _ in range(3):
    y = y * (half * y * y + 1.5)
  return y


def _load_edges(e_hbm, row, buf, base, sem):
  handles = [
      pltpu.async_copy(e_hbm.at[row, pl.ds(base + j * CHUNK, CHUNK)],
                       buf.at[j], sem)
      for j in range(FULL_CHUNKS)
  ]
  handles.append(
      pltpu.async_copy(e_hbm.at[row, pl.ds(base + FULL_CHUNKS * CHUNK, REM)],
                       buf.at[FULL_CHUNKS, pl.ds(0, REM)], sem))
  iota = lax.iota(jnp.int32, 16)
  group = 0
  for k in range(REM // 16, CHUNK // 16):
    buf[FULL_CHUNKS, pl.ds(16 * k, 16)] = N + 16 * group + iota
    group += 1
  for k in range(CHUNK // 16):
    buf[NCHUNK - 1, pl.ds(16 * k, 16)] = N + 16 * group + iota
    group += 1
  return handles


def _edge_loop(do_gather, dummy_hbm, featsh, acc, srcv, dstv, rowsv,
               gsem, ssem_a, ssem_b):
  ssems = (ssem_a, ssem_b)
  NG = NCHUNK // 8

  def _drain(b, sem):
    pltpu.make_async_copy(dummy_hbm.at[pl.ds(0, CHUNK)], rowsv.at[b],
                          sem).wait()

  if do_gather:
    for b in range(8):
      pltpu.async_copy(featsh.at[srcv.at[b]], rowsv.at[b], gsem)

  def pair(g2, carry):
    for p in range(2):
      g = g2 * 2 + p
      po = 8 * p
      qo = 8 * (1 - p)
      if do_gather:
        for b in range(8):
          _drain(po + b, gsem)

      @pl.when(g > 0)
      def _():
        for b in range(8):
          _drain(qo + b, ssems[1 - p])

      if do_gather:
        @pl.when(g + 1 < NG)
        def _():
          for b in range(8):
            pltpu.async_copy(featsh.at[srcv.at[(g + 1) * 8 + b]],
                             rowsv.at[qo + b], gsem)

      for b in range(8):
        pltpu.async_copy(rowsv.at[po + b], acc.at[dstv.at[g * 8 + b]],
                         ssems[p], add=True)
    return carry

  lax.fori_loop(0, NG // 2, pair, 0)
  last = (NG - 1) % 2
  for b in range(8):
    _drain(8 * last + b, ssems[last])


def _tile_ids():
  c = lax.axis_index("c")
  s = lax.axis_index("s")
  return c, s, s * 2 + c, s * ROWS_PER_TILE


def _drain_out(acc, tmpv, out_hbm, c, r0):
  plsc.subcore_barrier()
  pltpu.sync_copy(acc.at[pl.ds(r0, ROWS_PER_TILE)], tmpv)
  pltpu.sync_copy(tmpv, out_hbm.at[c, pl.ds(r0, ROWS_PER_TILE), :])


def _deg_body(ones_hbm, e_hbm, out_hbm, dstv, rowsv, tmpv, acc, featsh,
              gsem, ssem_a, ssem_b):
  c, s, wid, r0 = _tile_ids()
  handles = _load_edges(e_hbm, 1, dstv, wid * PER_TILE, gsem)
  handles += [
      pltpu.async_copy(ones_hbm.at[pl.ds(0, CHUNK)], rowsv.at[b], ssem_a)
      for b in range(16)
  ]
  pltpu.sync_copy(ones_hbm.at[pl.ds(r0, ROWS_PER_TILE)], tmpv)
  pltpu.sync_copy(tmpv, acc.at[pl.ds(r0, ROWS_PER_TILE)])
  for h in handles:
    h.wait()
  plsc.subcore_barrier()
  _edge_loop(False, ones_hbm, featsh, acc, None, dstv, rowsv,
             gsem, ssem_a, ssem_b)
  _drain_out(acc, tmpv, out_hbm, c, r0)


_sc_deg = functools.partial(
    pl.kernel,
    out_type=jax.ShapeDtypeStruct((2, NP, D_HID), jnp.float32),
    mesh=_mesh,
    scratch_types=[
        pltpu.VMEM((NCHUNK, CHUNK), jnp.int32),
        pltpu.VMEM((16, CHUNK, D_HID), jnp.float32),
        pltpu.VMEM((ROWS_PER_TILE, D_HID), jnp.float32),
        pltpu.VMEM_SHARED((NP, D_HID), jnp.float32),
        pltpu.VMEM_SHARED((NP, D_HID), jnp.float32),
        pltpu.SemaphoreType.DMA,
        pltpu.SemaphoreType.DMA,
        pltpu.SemaphoreType.DMA,
    ],
    compiler_params=_params,
)(_deg_body)


def _prop1_body(p1_hbm, degp_hbm, e_hbm, out_hbm,
                srcv, dstv, rowsv, av, bv, acc, featsh,
                gsem, ssem_a, ssem_b):
  c, s, wid, r0 = _tile_ids()
  handles = _load_edges(e_hbm, 1, dstv, wid * PER_TILE, gsem)
  handles += _load_edges(e_hbm, 0, srcv, wid * PER_TILE, gsem)
  pltpu.sync_copy(degp_hbm.at[0, pl.ds(r0, ROWS_PER_TILE), :], av)
  pltpu.sync_copy(degp_hbm.at[1, pl.ds(r0, ROWS_PER_TILE), :], bv)

  def dinv_loop(j4, carry):
    for k in range(4):
      j = j4 * 4 + k
      av[j] = _newton_rsqrt(av[j] + bv[j] - 1.0)
    return carry

  lax.fori_loop(0, ROWS_PER_TILE // 4, dinv_loop, 0)
  pltpu.sync_copy(p1_hbm.at[pl.ds(r0, ROWS_PER_TILE)], bv)

  def scale(j4, carry):
    for k in range(4):
      j = j4 * 4 + k
      bv[j] = bv[j] * av[j]
    return carry

  lax.fori_loop(0, ROWS_PER_TILE // 4, scale, 0)
  pltpu.sync_copy(bv, acc.at[pl.ds(r0, ROWS_PER_TILE)])
  pltpu.sync_copy(bv, featsh.at[pl.ds(r0, ROWS_PER_TILE)])
  for h in handles:
    h.wait()
  plsc.subcore_barrier()
  _edge_loop(True, p1_hbm, featsh, acc, srcv, dstv, rowsv,
             gsem, ssem_a, ssem_b)
  _drain_out(acc, av, out_hbm, c, r0)


_sc_prop1 = functools.partial(
    pl.kernel,
    out_type=jax.ShapeDtypeStruct((2, NP, D_HID), jnp.float32),
    mesh=_mesh,
    scratch_types=[
        pltpu.VMEM((NCHUNK, CHUNK), jnp.int32),
        pltpu.VMEM((NCHUNK, CHUNK), jnp.int32),
        pltpu.VMEM((16, CHUNK, D_HID), jnp.float32),
        pltpu.VMEM((ROWS_PER_TILE, D_HID), jnp.float32),
        pltpu.VMEM((ROWS_PER_TILE, D_HID), jnp.float32),
        pltpu.VMEM_SHARED((NP, D_HID), jnp.float32),
        pltpu.VMEM_SHARED((NP, D_HID), jnp.float32),
        pltpu.SemaphoreType.DMA,
        pltpu.SemaphoreType.DMA,
        pltpu.SemaphoreType.DMA,
    ],
    compiler_params=_params,
)(_prop1_body)


def _prop2_body(m1_hbm, p1_hbm, degp_hbm, b1_hbm, e_hbm, m2_hbm, hp_hbm,
                srcv, dstv, rowsv, av, bv, cv, dv, b1v,
                acc, featsh, gsem, ssem_a, ssem_b):
  c, s, wid, r0 = _tile_ids()
  handles = _load_edges(e_hbm, 1, dstv, wid * PER_TILE, gsem)
  handles += _load_edges(e_hbm, 0, srcv, wid * PER_TILE, gsem)
  pltpu.sync_copy(degp_hbm.at[0, pl.ds(r0, ROWS_PER_TILE), :], av)
  pltpu.sync_copy(degp_hbm.at[1, pl.ds(r0, ROWS_PER_TILE), :], bv)

  def dinv_loop(j4, carry):
    for k in range(4):
      j = j4 * 4 + k
      av[j] = _newton_rsqrt(av[j] + bv[j] - 1.0)
    return carry

  lax.fori_loop(0, ROWS_PER_TILE // 4, dinv_loop, 0)
  pltpu.sync_copy(p1_hbm.at[pl.ds(r0, ROWS_PER_TILE)], bv)
  pltpu.sync_copy(m1_hbm.at[0, pl.ds(r0, ROWS_PER_TILE), :], cv)
  pltpu.sync_copy(m1_hbm.at[1, pl.ds(r0, ROWS_PER_TILE), :], dv)
  pltpu.sync_copy(b1_hbm, b1v)
  b1row = b1v[0]

  def mid(j4, carry):
    for k in range(4):
      j = j4 * 4 + k
      dinv = av[j]
      m = cv[j] + dv[j] - bv[j] * dinv
      h = jnp.maximum(m * dinv + b1row, 0.0)
      dv[j] = h * dinv
    return carry

  lax.fori_loop(0, ROWS_PER_TILE // 4, mid, 0)
  pltpu.sync_copy(dv, acc.at[pl.ds(r0, ROWS_PER_TILE)])
  pltpu.sync_copy(dv, featsh.at[pl.ds(r0, ROWS_PER_TILE)])
  pltpu.sync_copy(dv, hp_hbm.at[pl.ds(r0, ROWS_PER_TILE)])
  for h in handles:
    h.wait()
  plsc.subcore_barrier()
  _edge_loop(True, p1_hbm, featsh, acc, srcv, dstv, rowsv,
             gsem, ssem_a, ssem_b)
  _drain_out(acc, dv, m2_hbm, c, r0)


_sc_prop2 = functools.partial(
    pl.kernel,
    out_type=(jax.ShapeDtypeStruct((2, NP, D_HID), jnp.float32),
              jax.ShapeDtypeStruct((NP, D_HID), jnp.float32)),
    mesh=_mesh,
    scratch_types=[
        pltpu.VMEM((NCHUNK, CHUNK), jnp.int32),
        pltpu.VMEM((NCHUNK, CHUNK), jnp.int32),
        pltpu.VMEM((16, CHUNK, D_HID), jnp.float32),
        pltpu.VMEM((ROWS_PER_TILE, D_HID), jnp.float32),
        pltpu.VMEM((ROWS_PER_TILE, D_HID), jnp.float32),
        pltpu.VMEM((ROWS_PER_TILE, D_HID), jnp.float32),
        pltpu.VMEM((ROWS_PER_TILE, D_HID), jnp.float32),
        pltpu.VMEM((1, D_HID), jnp.float32),
        pltpu.VMEM_SHARED((NP, D_HID), jnp.float32),
        pltpu.VMEM_SHARED((NP, D_HID), jnp.float32),
        pltpu.SemaphoreType.DMA,
        pltpu.SemaphoreType.DMA,
        pltpu.SemaphoreType.DMA,
    ],
    compiler_params=_params,
)(_prop2_body)


def _tc_matmul(x_ref, w1_ref, out_ref):
  out_ref[...] = jnp.dot(x_ref[...], w1_ref[...],
                         preferred_element_type=jnp.float32)


def _tc_final(m_ref, hp_ref, deg_ref, w2_ref, b2_ref, out_ref):
  dinv = lax.rsqrt(deg_ref[0] + deg_ref[1] - 1.0)
  m = ((m_ref[0] + m_ref[1] - hp_ref[...]) * dinv)[:N]
  out_ref[...] = jnp.dot(m, w2_ref[...],
                         preferred_element_type=jnp.float32) + b2_ref[...]


def kernel(V, E, X, W1, b1, W2, b2):
  del V
  f32 = jnp.float32
  ones = jnp.ones((NP, D_HID), f32)
  Xp = jnp.concatenate([X, jnp.zeros((NP - N, D_IN), f32)])

  degp = _sc_deg(ones, E)
  p1 = pl.pallas_call(
      _tc_matmul,
      out_shape=jax.ShapeDtypeStruct((NP, D_HID), f32),
  )(Xp, W1)

  m1 = _sc_prop1(p1, degp, E)
  m2, hp = _sc_prop2(m1, p1, degp, b1.reshape(1, D_HID), E)

  out = pl.pallas_call(
      _tc_final,
      out_shape=jax.ShapeDtypeStruct((N, D_OUT), f32),
  )(m2, hp, degp, W2, b2.reshape(1, D_OUT))
  return out


# speedup vs baseline: 1.3213x; 1.0233x over previous
import functools

import jax
import jax.numpy as jnp
from jax import lax
from jax.experimental import pallas as pl
from jax.experimental.pallas import tpu as pltpu
from jax.experimental.pallas import tpu_sc as plsc

N = 10000
NP = 10240
D_IN = 128
D_HID = 16
D_OUT = 128
E_REAL = 320000
CHUNK = 128
NCHUNK = 80
PER_TILE = E_REAL // 32
FULL_CHUNKS = PER_TILE // CHUNK
REM = PER_TILE - FULL_CHUNKS * CHUNK
ROWS_PER_TILE = NP // 16

_mesh = plsc.VectorSubcoreMesh(core_axis_name="c", subcore_axis_name="s")
_params = pltpu.CompilerParams(use_tc_tiling_on_sc=False)


def _newton_rsqrt(x):
  i = lax.bitcast_convert_type(x, jnp.int32)
  i = jnp.full((16,), 0x5F3759DF, jnp.int32) - lax.shift_right_arithmetic(
      i, jnp.full((16,), 1, jnp.int32))
  y = lax.bitcast_convert_type(i, jnp.float32)
  half = x * (-0.5)
  for _ in range(3):
    y = y * (half * y * y + 1.5)
  return y


def _load_edges(e_hbm, row, buf, base, sem):
  handles = [
      pltpu.async_copy(e_hbm.at[row, pl.ds(base + j * CHUNK, CHUNK)],
                       buf.at[j], sem)
      for j in range(FULL_CHUNKS)
  ]
  handles.append(
      pltpu.async_copy(e_hbm.at[row, pl.ds(base + FULL_CHUNKS * CHUNK, REM)],
                       buf.at[FULL_CHUNKS, pl.ds(0, REM)], sem))
  iota = lax.iota(jnp.int32, 16)
  group = 0
  for k in range(REM // 16, CHUNK // 16):
    buf[FULL_CHUNKS, pl.ds(16 * k, 16)] = N + 16 * group + iota
    group += 1
  for k in range(CHUNK // 16):
    buf[NCHUNK - 1, pl.ds(16 * k, 16)] = N + 16 * group + iota
    group += 1
  return handles


def _edge_loop(do_gather, dummy_hbm, featsh, acc, srcv, dstv, rowsv,
               gsem, ssem_a, ssem_b):
  ssems = (ssem_a, ssem_b)
  NG = NCHUNK // 8

  def _drain(b, sem):
    pltpu.make_async_copy(dummy_hbm.at[pl.ds(0, CHUNK)], rowsv.at[b],
                          sem).wait()

  if do_gather:
    for b in range(8):
      pltpu.async_copy(featsh.at[srcv.at[b]], rowsv.at[b], gsem)

  def pair(g2, carry):
    for p in range(2):
      g = g2 * 2 + p
      po = 8 * p
      qo = 8 * (1 - p)
      if do_gather:
        for b in range(8):
          _drain(po + b, gsem)

      @pl.when(g > 0)
      def _():
        for b in range(8):
          _drain(qo + b, ssems[1 - p])

      if do_gather:
        @pl.when(g + 1 < NG)
        def _():
          for b in range(8):
            pltpu.async_copy(featsh.at[srcv.at[(g + 1) * 8 + b]],
                             rowsv.at[qo + b], gsem)

      for b in range(8):
        pltpu.async_copy(rowsv.at[po + b], acc.at[dstv.at[g * 8 + b]],
                         ssems[p], add=True)
    return carry

  def pair_paced(g2, carry):
    for p in range(2):
      g = g2 * 2 + p
      po = 8 * p
      for b in range(8):
        pltpu.async_copy(rowsv.at[po + b], acc.at[dstv.at[g * 8 + b]],
                         ssems[p], add=True)
      for b in range(8):
        _drain(po + b, ssems[p])
    return carry

  lax.fori_loop(0, NG // 2, pair if do_gather else pair_paced, 0)
  if do_gather:
    last = (NG - 1) % 2
    for b in range(8):
      _drain(8 * last + b, ssems[last])


def _tile_ids():
  c = lax.axis_index("c")
  s = lax.axis_index("s")
  return c, s, s * 2 + c, s * ROWS_PER_TILE


def _drain_out(acc, tmpv, out_hbm, c, r0):
  plsc.subcore_barrier()
  pltpu.sync_copy(acc.at[pl.ds(r0, ROWS_PER_TILE)], tmpv)
  pltpu.sync_copy(tmpv, out_hbm.at[c, pl.ds(r0, ROWS_PER_TILE), :])


def _deg_body(ones_hbm, e_hbm, out_hbm, dstv, rowsv, tmpv, acc, featsh,
              gsem, ssem_a, ssem_b):
  c, s, wid, r0 = _tile_ids()
  handles = _load_edges(e_hbm, 1, dstv, wid * PER_TILE, gsem)
  handles += [
      pltpu.async_copy(ones_hbm.at[pl.ds(0, CHUNK)], rowsv.at[b], ssem_a)
      for b in range(16)
  ]
  pltpu.sync_copy(ones_hbm.at[pl.ds(r0, ROWS_PER_TILE)], tmpv)
  pltpu.sync_copy(tmpv, acc.at[pl.ds(r0, ROWS_PER_TILE)])
  for h in handles:
    h.wait()
  plsc.subcore_barrier()
  _edge_loop(False, ones_hbm, featsh, acc, None, dstv, rowsv,
             gsem, ssem_a, ssem_b)
  _drain_out(acc, tmpv, out_hbm, c, r0)


_sc_deg = functools.partial(
    pl.kernel,
    out_type=jax.ShapeDtypeStruct((2, NP, D_HID), jnp.float32),
    mesh=_mesh,
    scratch_types=[
        pltpu.VMEM((NCHUNK, CHUNK), jnp.int32),
        pltpu.VMEM((16, CHUNK, D_HID), jnp.float32),
        pltpu.VMEM((ROWS_PER_TILE, D_HID), jnp.float32),
        pltpu.VMEM_SHARED((NP, D_HID), jnp.float32),
        pltpu.VMEM_SHARED((NP, D_HID), jnp.float32),
        pltpu.SemaphoreType.DMA,
        pltpu.SemaphoreType.DMA,
        pltpu.SemaphoreType.DMA,
    ],
    compiler_params=_params,
)(_deg_body)


def _prop1_body(p1_hbm, degp_hbm, e_hbm, out_hbm,
                srcv, dstv, rowsv, av, bv, acc, featsh,
                gsem, ssem_a, ssem_b):
  c, s, wid, r0 = _tile_ids()
  handles = _load_edges(e_hbm, 1, dstv, wid * PER_TILE, gsem)
  handles += _load_edges(e_hbm, 0, srcv, wid * PER_TILE, gsem)
  pltpu.sync_copy(degp_hbm.at[0, pl.ds(r0, ROWS_PER_TILE), :], av)
  pltpu.sync_copy(degp_hbm.at[1, pl.ds(r0, ROWS_PER_TILE), :], bv)

  def dinv_loop(j4, carry):
    for k in range(4):
      j = j4 * 4 + k
      av[j] = _newton_rsqrt(av[j] + bv[j] - 1.0)
    return carry

  lax.fori_loop(0, ROWS_PER_TILE // 4, dinv_loop, 0)
  pltpu.sync_copy(p1_hbm.at[pl.ds(r0, ROWS_PER_TILE)], bv)

  def scale(j4, carry):
    for k in range(4):
      j = j4 * 4 + k
      bv[j] = bv[j] * av[j]
    return carry

  lax.fori_loop(0, ROWS_PER_TILE // 4, scale, 0)
  pltpu.sync_copy(bv, acc.at[pl.ds(r0, ROWS_PER_TILE)])
  pltpu.sync_copy(bv, featsh.at[pl.ds(r0, ROWS_PER_TILE)])
  for h in handles:
    h.wait()
  plsc.subcore_barrier()
  _edge_loop(True, p1_hbm, featsh, acc, srcv, dstv, rowsv,
             gsem, ssem_a, ssem_b)
  _drain_out(acc, av, out_hbm, c, r0)


_sc_prop1 = functools.partial(
    pl.kernel,
    out_type=jax.ShapeDtypeStruct((2, NP, D_HID), jnp.float32),
    mesh=_mesh,
    scratch_types=[
        pltpu.VMEM((NCHUNK, CHUNK), jnp.int32),
        pltpu.VMEM((NCHUNK, CHUNK), jnp.int32),
        pltpu.VMEM((16, CHUNK, D_HID), jnp.float32),
        pltpu.VMEM((ROWS_PER_TILE, D_HID), jnp.float32),
        pltpu.VMEM((ROWS_PER_TILE, D_HID), jnp.float32),
        pltpu.VMEM_SHARED((NP, D_HID), jnp.float32),
        pltpu.VMEM_SHARED((NP, D_HID), jnp.float32),
        pltpu.SemaphoreType.DMA,
        pltpu.SemaphoreType.DMA,
        pltpu.SemaphoreType.DMA,
    ],
    compiler_params=_params,
)(_prop1_body)


def _prop2_body(m1_hbm, p1_hbm, degp_hbm, b1_hbm, e_hbm, m2_hbm, hp_hbm,
                srcv, dstv, rowsv, av, bv, cv, dv, b1v,
                acc, featsh, gsem, ssem_a, ssem_b):
  c, s, wid, r0 = _tile_ids()
  handles = _load_edges(e_hbm, 1, dstv, wid * PER_TILE, gsem)
  handles += _load_edges(e_hbm, 0, srcv, wid * PER_TILE, gsem)
  pltpu.sync_copy(degp_hbm.at[0, pl.ds(r0, ROWS_PER_TILE), :], av)
  pltpu.sync_copy(degp_hbm.at[1, pl.ds(r0, ROWS_PER_TILE), :], bv)

  def dinv_loop(j4, carry):
    for k in range(4):
      j = j4 * 4 + k
      av[j] = _newton_rsqrt(av[j] + bv[j] - 1.0)
    return carry

  lax.fori_loop(0, ROWS_PER_TILE // 4, dinv_loop, 0)
  pltpu.sync_copy(p1_hbm.at[pl.ds(r0, ROWS_PER_TILE)], bv)
  pltpu.sync_copy(m1_hbm.at[0, pl.ds(r0, ROWS_PER_TILE), :], cv)
  pltpu.sync_copy(m1_hbm.at[1, pl.ds(r0, ROWS_PER_TILE), :], dv)
  pltpu.sync_copy(b1_hbm, b1v)
  b1row = b1v[0]

  def mid(j4, carry):
    for k in range(4):
      j = j4 * 4 + k
      dinv = av[j]
      m = cv[j] + dv[j] - bv[j] * dinv
      h = jnp.maximum(m * dinv + b1row, 0.0)
      dv[j] = h * dinv
    return carry

  lax.fori_loop(0, ROWS_PER_TILE // 4, mid, 0)
  pltpu.sync_copy(dv, acc.at[pl.ds(r0, ROWS_PER_TILE)])
  pltpu.sync_copy(dv, featsh.at[pl.ds(r0, ROWS_PER_TILE)])
  for h in handles:
    h.wait()
  plsc.subcore_barrier()
  _edge_loop(True, p1_hbm, featsh, acc, srcv, dstv, rowsv,
             gsem, ssem_a, ssem_b)
  plsc.subcore_barrier()
  pltpu.sync_copy(acc.at[pl.ds(r0, ROWS_PER_TILE)], cv)

  def fin(j4, carry):
    for k in range(4):
      j = j4 * 4 + k
      cv[j] = cv[j] * av[j]
      dv[j] = dv[j] * av[j]
    return carry

  lax.fori_loop(0, ROWS_PER_TILE // 4, fin, 0)
  pltpu.sync_copy(cv, m2_hbm.at[c, pl.ds(r0, ROWS_PER_TILE), :])
  pltpu.sync_copy(dv, hp_hbm.at[pl.ds(r0, ROWS_PER_TILE)])


_sc_prop2 = functools.partial(
    pl.kernel,
    out_type=(jax.ShapeDtypeStruct((2, NP, D_HID), jnp.float32),
              jax.ShapeDtypeStruct((NP, D_HID), jnp.float32)),
    mesh=_mesh,
    scratch_types=[
        pltpu.VMEM((NCHUNK, CHUNK), jnp.int32),
        pltpu.VMEM((NCHUNK, CHUNK), jnp.int32),
        pltpu.VMEM((16, CHUNK, D_HID), jnp.float32),
        pltpu.VMEM((ROWS_PER_TILE, D_HID), jnp.float32),
        pltpu.VMEM((ROWS_PER_TILE, D_HID), jnp.float32),
        pltpu.VMEM((ROWS_PER_TILE, D_HID), jnp.float32),
        pltpu.VMEM((ROWS_PER_TILE, D_HID), jnp.float32),
        pltpu.VMEM((1, D_HID), jnp.float32),
        pltpu.VMEM_SHARED((NP, D_HID), jnp.float32),
        pltpu.VMEM_SHARED((NP, D_HID), jnp.float32),
        pltpu.SemaphoreType.DMA,
        pltpu.SemaphoreType.DMA,
        pltpu.SemaphoreType.DMA,
    ],
    compiler_params=_params,
)(_prop2_body)


def _tc_matmul(x_ref, w1_ref, out_ref):
  out_ref[...] = jnp.dot(x_ref[...], w1_ref[...],
                         preferred_element_type=jnp.float32)


def _tc_final(m_ref, hpd_ref, w2_ref, b2_ref, out_ref):
  m = (m_ref[0] + m_ref[1] - hpd_ref[...])[:N]
  out_ref[...] = jnp.dot(m, w2_ref[...],
                         preferred_element_type=jnp.float32) + b2_ref[...]


def kernel(V, E, X, W1, b1, W2, b2):
  del V
  f32 = jnp.float32
  ones = jnp.ones((NP, D_HID), f32)
  Xp = jnp.concatenate([X, jnp.zeros((NP - N, D_IN), f32)])

  degp = _sc_deg(ones, E)
  p1 = pl.pallas_call(
      _tc_matmul,
      out_shape=jax.ShapeDtypeStruct((NP, D_HID), f32),
  )(Xp, W1)

  m1 = _sc_prop1(p1, degp, E)
  m2, hpd = _sc_prop2(m1, p1, degp, b1.reshape(1, D_HID), E)

  out = pl.pallas_call(
      _tc_final,
      out_shape=jax.ShapeDtypeStruct((N, D_OUT), f32),
  )(m2, hpd, W2, b2.reshape(1, D_OUT))
  return out


# speedup vs baseline: 1.3486x; 1.0207x over previous
import functools

import jax
import jax.numpy as jnp
from jax import lax
from jax.experimental import pallas as pl
from jax.experimental.pallas import tpu as pltpu
from jax.experimental.pallas import tpu_sc as plsc

N = 10000
NP = 10240
D_IN = 128
D_HID = 16
D_OUT = 128
E_REAL = 320000
CHUNK = 128
NCHUNK = 80
PER_TILE = E_REAL // 32
FULL_CHUNKS = PER_TILE // CHUNK
REM = PER_TILE - FULL_CHUNKS * CHUNK
ROWS_PER_TILE = NP // 16

_mesh = plsc.VectorSubcoreMesh(core_axis_name="c", subcore_axis_name="s")
_params = pltpu.CompilerParams(use_tc_tiling_on_sc=False)


def _newton_rsqrt(x):
  i = lax.bitcast_convert_type(x, jnp.int32)
  i = jnp.full((16,), 0x5F3759DF, jnp.int32) - lax.shift_right_arithmetic(
      i, jnp.full((16,), 1, jnp.int32))
  y = lax.bitcast_convert_type(i, jnp.float32)
  half = x * (-0.5)
  for _ in range(3):
    y = y * (half * y * y + 1.5)
  return y


def _load_edges(e_hbm, row, buf, base, sem):
  def ld(j, carry):
    pltpu.async_copy(e_hbm.at[row, pl.ds(base + j * CHUNK, CHUNK)],
                     buf.at[j], sem)
    return carry

  lax.fori_loop(0, FULL_CHUNKS, ld, 0)
  pltpu.sync_copy(e_hbm.at[row, pl.ds(base + FULL_CHUNKS * CHUNK, REM)],
                  buf.at[FULL_CHUNKS, pl.ds(0, REM)])
  iota = lax.iota(jnp.int32, 16)
  group = 0
  for k in range(REM // 16, CHUNK // 16):
    buf[FULL_CHUNKS, pl.ds(16 * k, 16)] = N + 16 * group + iota
    group += 1
  for k in range(CHUNK // 16):
    buf[NCHUNK - 1, pl.ds(16 * k, 16)] = N + 16 * group + iota
    group += 1

  def drain():
    def wt(j, carry):
      pltpu.make_async_copy(e_hbm.at[row, pl.ds(base, CHUNK)],
                            buf.at[j], sem).wait()
      return carry
    lax.fori_loop(0, FULL_CHUNKS, wt, 0)

  return [drain]


def _edge_loop(do_gather, dummy_hbm, featsh, acc, srcv, dstv, rowsv,
               gsem, ssem_a, ssem_b):
  ssems = (ssem_a, ssem_b)
  NG = NCHUNK // 8

  def _drain(b, sem):
    pltpu.make_async_copy(dummy_hbm.at[pl.ds(0, CHUNK)], rowsv.at[b],
                          sem).wait()

  if do_gather:
    for b in range(8):
      pltpu.async_copy(featsh.at[srcv.at[b]], rowsv.at[b], gsem)

  def pair(g2, carry):
    for p in range(2):
      g = g2 * 2 + p
      po = 8 * p
      qo = 8 * (1 - p)
      if do_gather:
        for b in range(8):
          _drain(po + b, gsem)

      @pl.when(g > 0)
      def _():
        for b in range(8):
          _drain(qo + b, ssems[1 - p])

      if do_gather:
        @pl.when(g + 1 < NG)
        def _():
          for b in range(8):
            pltpu.async_copy(featsh.at[srcv.at[(g + 1) * 8 + b]],
                             rowsv.at[qo + b], gsem)

      for b in range(8):
        pltpu.async_copy(rowsv.at[po + b], acc.at[dstv.at[g * 8 + b]],
                         ssems[p], add=True)
    return carry

  def pair_paced(g2, carry):
    for p in range(2):
      g = g2 * 2 + p
      po = 8 * p
      for b in range(8):
        pltpu.async_copy(rowsv.at[po + b], acc.at[dstv.at[g * 8 + b]],
                         ssems[p], add=True)
      for b in range(8):
        _drain(po + b, ssems[p])
    return carry

  lax.fori_loop(0, NG // 2, pair if do_gather else pair_paced, 0)
  if do_gather:
    last = (NG - 1) % 2
    for b in range(8):
      _drain(8 * last + b, ssems[last])


def _tile_ids():
  c = lax.axis_index("c")
  s = lax.axis_index("s")
  return c, s, s * 2 + c, s * ROWS_PER_TILE


def _drain_out(acc, tmpv, out_hbm, c, r0):
  plsc.subcore_barrier()
  pltpu.sync_copy(acc.at[pl.ds(r0, ROWS_PER_TILE)], tmpv)
  pltpu.sync_copy(tmpv, out_hbm.at[c, pl.ds(r0, ROWS_PER_TILE), :])


def _deg_body(ones_hbm, e_hbm, out_hbm, dstv, rowsv, tmpv, acc, featsh,
              gsem, ssem_a, ssem_b):
  c, s, wid, r0 = _tile_ids()
  handles = _load_edges(e_hbm, 1, dstv, wid * PER_TILE, gsem)
  handles += [
      pltpu.async_copy(ones_hbm.at[pl.ds(0, CHUNK)], rowsv.at[b], ssem_a).wait
      for b in range(16)
  ]
  pltpu.sync_copy(ones_hbm.at[pl.ds(r0, ROWS_PER_TILE)], tmpv)
  pltpu.sync_copy(tmpv, acc.at[pl.ds(r0, ROWS_PER_TILE)])
  for h in handles:
    h()
  plsc.subcore_barrier()
  _edge_loop(False, ones_hbm, featsh, acc, None, dstv, rowsv,
             gsem, ssem_a, ssem_b)
  _drain_out(acc, tmpv, out_hbm, c, r0)


_sc_deg = functools.partial(
    pl.kernel,
    out_type=jax.ShapeDtypeStruct((2, NP, D_HID), jnp.float32),
    mesh=_mesh,
    scratch_types=[
        pltpu.VMEM((NCHUNK, CHUNK), jnp.int32),
        pltpu.VMEM((16, CHUNK, D_HID), jnp.float32),
        pltpu.VMEM((ROWS_PER_TILE, D_HID), jnp.float32),
        pltpu.VMEM_SHARED((NP, D_HID), jnp.float32),
        pltpu.VMEM_SHARED((NP, D_HID), jnp.float32),
        pltpu.SemaphoreType.DMA,
        pltpu.SemaphoreType.DMA,
        pltpu.SemaphoreType.DMA,
    ],
    compiler_params=_params,
)(_deg_body)


def _prop1_body(p1_hbm, degp_hbm, e_hbm, out_hbm,
                srcv, dstv, rowsv, av, bv, acc, featsh,
                gsem, ssem_a, ssem_b):
  c, s, wid, r0 = _tile_ids()
  handles = _load_edges(e_hbm, 1, dstv, wid * PER_TILE, gsem)
  handles += _load_edges(e_hbm, 0, srcv, wid * PER_TILE, gsem)
  pltpu.sync_copy(degp_hbm.at[0, pl.ds(r0, ROWS_PER_TILE), :], av)
  pltpu.sync_copy(degp_hbm.at[1, pl.ds(r0, ROWS_PER_TILE), :], bv)

  def dinv_loop(j4, carry):
    for k in range(4):
      j = j4 * 4 + k
      av[j] = _newton_rsqrt(av[j] + bv[j] - 1.0)
    return carry

  lax.fori_loop(0, ROWS_PER_TILE // 4, dinv_loop, 0)
  pltpu.sync_copy(p1_hbm.at[pl.ds(r0, ROWS_PER_TILE)], bv)

  def scale(j4, carry):
    for k in range(4):
      j = j4 * 4 + k
      bv[j] = bv[j] * av[j]
    return carry

  lax.fori_loop(0, ROWS_PER_TILE // 4, scale, 0)
  pltpu.sync_copy(bv, acc.at[pl.ds(r0, ROWS_PER_TILE)])
  pltpu.sync_copy(bv, featsh.at[pl.ds(r0, ROWS_PER_TILE)])
  for h in handles:
    h()
  plsc.subcore_barrier()
  _edge_loop(True, p1_hbm, featsh, acc, srcv, dstv, rowsv,
             gsem, ssem_a, ssem_b)
  _drain_out(acc, av, out_hbm, c, r0)


_sc_prop1 = functools.partial(
    pl.kernel,
    out_type=jax.ShapeDtypeStruct((2, NP, D_HID), jnp.float32),
    mesh=_mesh,
    scratch_types=[
        pltpu.VMEM((NCHUNK, CHUNK), jnp.int32),
        pltpu.VMEM((NCHUNK, CHUNK), jnp.int32),
        pltpu.VMEM((16, CHUNK, D_HID), jnp.float32),
        pltpu.VMEM((ROWS_PER_TILE, D_HID), jnp.float32),
        pltpu.VMEM((ROWS_PER_TILE, D_HID), jnp.float32),
        pltpu.VMEM_SHARED((NP, D_HID), jnp.float32),
        pltpu.VMEM_SHARED((NP, D_HID), jnp.float32),
        pltpu.SemaphoreType.DMA,
        pltpu.SemaphoreType.DMA,
        pltpu.SemaphoreType.DMA,
    ],
    compiler_params=_params,
)(_prop1_body)


def _prop2_body(m1_hbm, p1_hbm, degp_hbm, b1_hbm, e_hbm, m2_hbm, hp_hbm,
                srcv, dstv, rowsv, av, bv, cv, dv, b1v,
                acc, featsh, gsem, ssem_a, ssem_b):
  c, s, wid, r0 = _tile_ids()
  handles = _load_edges(e_hbm, 1, dstv, wid * PER_TILE, gsem)
  handles += _load_edges(e_hbm, 0, srcv, wid * PER_TILE, gsem)
  pltpu.sync_copy(degp_hbm.at[0, pl.ds(r0, ROWS_PER_TILE), :], av)
  pltpu.sync_copy(degp_hbm.at[1, pl.ds(r0, ROWS_PER_TILE), :], bv)

  def dinv_loop(j4, carry):
    for k in range(4):
      j = j4 * 4 + k
      av[j] = _newton_rsqrt(av[j] + bv[j] - 1.0)
    return carry

  lax.fori_loop(0, ROWS_PER_TILE // 4, dinv_loop, 0)
  pltpu.sync_copy(p1_hbm.at[pl.ds(r0, ROWS_PER_TILE)], bv)
  pltpu.sync_copy(m1_hbm.at[0, pl.ds(r0, ROWS_PER_TILE), :], cv)
  pltpu.sync_copy(m1_hbm.at[1, pl.ds(r0, ROWS_PER_TILE), :], dv)
  pltpu.sync_copy(b1_hbm, b1v)
  b1row = b1v[0]

  def mid(j4, carry):
    for k in range(4):
      j = j4 * 4 + k
      dinv = av[j]
      m = cv[j] + dv[j] - bv[j] * dinv
      h = jnp.maximum(m * dinv + b1row, 0.0)
      dv[j] = h * dinv
    return carry

  lax.fori_loop(0, ROWS_PER_TILE // 4, mid, 0)
  pltpu.sync_copy(dv, acc.at[pl.ds(r0, ROWS_PER_TILE)])
  pltpu.sync_copy(dv, featsh.at[pl.ds(r0, ROWS_PER_TILE)])
  for h in handles:
    h()
  plsc.subcore_barrier()
  _edge_loop(True, p1_hbm, featsh, acc, srcv, dstv, rowsv,
             gsem, ssem_a, ssem_b)
  plsc.subcore_barrier()
  pltpu.sync_copy(acc.at[pl.ds(r0, ROWS_PER_TILE)], cv)

  def fin(j4, carry):
    for k in range(4):
      j = j4 * 4 + k
      cv[j] = cv[j] * av[j]
      dv[j] = dv[j] * av[j]
    return carry

  lax.fori_loop(0, ROWS_PER_TILE // 4, fin, 0)
  pltpu.sync_copy(cv, m2_hbm.at[c, pl.ds(r0, ROWS_PER_TILE), :])
  pltpu.sync_copy(dv, hp_hbm.at[pl.ds(r0, ROWS_PER_TILE)])


_sc_prop2 = functools.partial(
    pl.kernel,
    out_type=(jax.ShapeDtypeStruct((2, NP, D_HID), jnp.float32),
              jax.ShapeDtypeStruct((NP, D_HID), jnp.float32)),
    mesh=_mesh,
    scratch_types=[
        pltpu.VMEM((NCHUNK, CHUNK), jnp.int32),
        pltpu.VMEM((NCHUNK, CHUNK), jnp.int32),
        pltpu.VMEM((16, CHUNK, D_HID), jnp.float32),
        pltpu.VMEM((ROWS_PER_TILE, D_HID), jnp.float32),
        pltpu.VMEM((ROWS_PER_TILE, D_HID), jnp.float32),
        pltpu.VMEM((ROWS_PER_TILE, D_HID), jnp.float32),
        pltpu.VMEM((ROWS_PER_TILE, D_HID), jnp.float32),
        pltpu.VMEM((1, D_HID), jnp.float32),
        pltpu.VMEM_SHARED((NP, D_HID), jnp.float32),
        pltpu.VMEM_SHARED((NP, D_HID), jnp.float32),
        pltpu.SemaphoreType.DMA,
        pltpu.SemaphoreType.DMA,
        pltpu.SemaphoreType.DMA,
    ],
    compiler_params=_params,
)(_prop2_body)


def _tc_matmul(x_ref, w1_ref, out_ref):
  out_ref[...] = jnp.dot(x_ref[...], w1_ref[...],
                         preferred_element_type=jnp.float32)


def _tc_final(m_ref, hpd_ref, w2_ref, b2_ref, out_ref):
  m = (m_ref[0] + m_ref[1] - hpd_ref[...])[:N]
  out_ref[...] = jnp.dot(m, w2_ref[...],
                         preferred_element_type=jnp.float32) + b2_ref[...]


def kernel(V, E, X, W1, b1, W2, b2):
  del V
  f32 = jnp.float32
  ones = jnp.ones((NP, D_HID), f32)
  Xp = jnp.concatenate([X, jnp.zeros((NP - N, D_IN), f32)])

  degp = _sc_deg(ones, E)
  p1 = pl.pallas_call(
      _tc_matmul,
      out_shape=jax.ShapeDtypeStruct((NP, D_HID), f32),
  )(Xp, W1)

  m1 = _sc_prop1(p1, degp, E)
  m2, hpd = _sc_prop2(m1, p1, degp, b1.reshape(1, D_HID), E)

  out = pl.pallas_call(
      _tc_final,
      out_shape=jax.ShapeDtypeStruct((N, D_OUT), f32),
  )(m2, hpd, W2, b2.reshape(1, D_OUT))
  return out


# speedup vs baseline: 1.3598x; 1.0083x over previous
import functools

import jax
import jax.numpy as jnp
from jax import lax
from jax.experimental import pallas as pl
from jax.experimental.pallas import tpu as pltpu
from jax.experimental.pallas import tpu_sc as plsc

N = 10000
NP = 10240
D_IN = 128
D_HID = 16
D_OUT = 128
E_REAL = 320000
CHUNK = 128
NCHUNK = 80
PER_TILE = E_REAL // 32
FULL_CHUNKS = PER_TILE // CHUNK
REM = PER_TILE - FULL_CHUNKS * CHUNK
ROWS_PER_TILE = NP // 16

_mesh = plsc.VectorSubcoreMesh(core_axis_name="c", subcore_axis_name="s")
_params = pltpu.CompilerParams(use_tc_tiling_on_sc=False)


def _newton_rsqrt(x):
  i = lax.bitcast_convert_type(x, jnp.int32)
  i = jnp.full((16,), 0x5F3759DF, jnp.int32) - lax.shift_right_arithmetic(
      i, jnp.full((16,), 1, jnp.int32))
  y = lax.bitcast_convert_type(i, jnp.float32)
  half = x * (-0.5)
  for _ in range(3):
    y = y * (half * y * y + 1.5)
  return y


ECHUNKS = E_REAL // CHUNK


def _load_edges(e3_hbm, row, buf, wid):
  b0 = (ECHUNKS * wid) // 32
  cnt = (ECHUNKS * (wid + 1)) // 32 - b0
  pltpu.sync_copy(e3_hbm.at[row, pl.ds(b0, 79), :], buf.at[pl.ds(0, 79)])
  iota = lax.iota(jnp.int32, 16)

  @pl.when(cnt == 78)
  def _():
    for k in range(8):
      buf[78, pl.ds(16 * k, 16)] = N + 16 * k + iota

  for k in range(8):
    buf[NCHUNK - 1, pl.ds(16 * k, 16)] = N + 16 * (k + 7) + iota


def _edge_loop(do_gather, dummy_hbm, featsh, acc, srcv, dstv, rowsv,
               gsem, ssem_a, ssem_b):
  ssems = (ssem_a, ssem_b)
  NG = NCHUNK // 8

  def _drain(b, sem):
    pltpu.make_async_copy(dummy_hbm.at[pl.ds(0, CHUNK)], rowsv.at[b],
                          sem).wait()

  if do_gather:
    for b in range(8):
      pltpu.async_copy(featsh.at[srcv.at[b]], rowsv.at[b], gsem)

  def pair(g2, carry):
    for p in range(2):
      g = g2 * 2 + p
      po = 8 * p
      qo = 8 * (1 - p)
      if do_gather:
        for b in range(8):
          _drain(po + b, gsem)

      @pl.when(g > 0)
      def _():
        for b in range(8):
          _drain(qo + b, ssems[1 - p])

      if do_gather:
        @pl.when(g + 1 < NG)
        def _():
          for b in range(8):
            pltpu.async_copy(featsh.at[srcv.at[(g + 1) * 8 + b]],
                             rowsv.at[qo + b], gsem)

      for b in range(8):
        pltpu.async_copy(rowsv.at[po + b], acc.at[dstv.at[g * 8 + b]],
                         ssems[p], add=True)
    return carry

  def pair_paced(g2, carry):
    for p in range(2):
      g = g2 * 2 + p
      po = 8 * p
      for b in range(8):
        pltpu.async_copy(rowsv.at[po + b], acc.at[dstv.at[g * 8 + b]],
                         ssems[p], add=True)
      for b in range(8):
        _drain(po + b, ssems[p])
    return carry

  lax.fori_loop(0, NG // 2, pair if do_gather else pair_paced, 0)
  if do_gather:
    last = (NG - 1) % 2
    for b in range(8):
      _drain(8 * last + b, ssems[last])


def _tile_ids():
  c = lax.axis_index("c")
  s = lax.axis_index("s")
  return c, s, s * 2 + c, s * ROWS_PER_TILE


def _drain_out(acc, tmpv, out_hbm, c, r0):
  plsc.subcore_barrier()
  pltpu.sync_copy(acc.at[pl.ds(r0, ROWS_PER_TILE)], tmpv)
  pltpu.sync_copy(tmpv, out_hbm.at[c, pl.ds(r0, ROWS_PER_TILE), :])


def _deg_body(ones_hbm, e_hbm, out_hbm, dstv, rowsv, tmpv, acc, featsh,
              gsem, ssem_a, ssem_b):
  c, s, wid, r0 = _tile_ids()
  _load_edges(e_hbm, 1, dstv, wid)
  handles = [
      pltpu.async_copy(ones_hbm.at[pl.ds(0, CHUNK)], rowsv.at[b], ssem_a).wait
      for b in range(16)
  ]
  pltpu.sync_copy(ones_hbm.at[pl.ds(r0, ROWS_PER_TILE)], tmpv)
  pltpu.sync_copy(tmpv, acc.at[pl.ds(r0, ROWS_PER_TILE)])
  for h in handles:
    h()
  plsc.subcore_barrier()
  _edge_loop(False, ones_hbm, featsh, acc, None, dstv, rowsv,
             gsem, ssem_a, ssem_b)
  _drain_out(acc, tmpv, out_hbm, c, r0)


_sc_deg = functools.partial(
    pl.kernel,
    out_type=jax.ShapeDtypeStruct((2, NP, D_HID), jnp.float32),
    mesh=_mesh,
    scratch_types=[
        pltpu.VMEM((NCHUNK, CHUNK), jnp.int32),
        pltpu.VMEM((16, CHUNK, D_HID), jnp.float32),
        pltpu.VMEM((ROWS_PER_TILE, D_HID), jnp.float32),
        pltpu.VMEM_SHARED((NP, D_HID), jnp.float32),
        pltpu.VMEM_SHARED((NP, D_HID), jnp.float32),
        pltpu.SemaphoreType.DMA,
        pltpu.SemaphoreType.DMA,
        pltpu.SemaphoreType.DMA,
    ],
    compiler_params=_params,
)(_deg_body)


def _prop1_body(p1_hbm, degp_hbm, e_hbm, out_hbm,
                srcv, dstv, rowsv, av, bv, acc, featsh,
                gsem, ssem_a, ssem_b):
  c, s, wid, r0 = _tile_ids()
  _load_edges(e_hbm, 1, dstv, wid)
  _load_edges(e_hbm, 0, srcv, wid)
  pltpu.sync_copy(degp_hbm.at[0, pl.ds(r0, ROWS_PER_TILE), :], av)
  pltpu.sync_copy(degp_hbm.at[1, pl.ds(r0, ROWS_PER_TILE), :], bv)

  def dinv_loop(j4, carry):
    for k in range(4):
      j = j4 * 4 + k
      av[j] = _newton_rsqrt(av[j] + bv[j] - 1.0)
    return carry

  lax.fori_loop(0, ROWS_PER_TILE // 4, dinv_loop, 0)
  pltpu.sync_copy(p1_hbm.at[pl.ds(r0, ROWS_PER_TILE)], bv)

  def scale(j4, carry):
    for k in range(4):
      j = j4 * 4 + k
      bv[j] = bv[j] * av[j]
    return carry

  lax.fori_loop(0, ROWS_PER_TILE // 4, scale, 0)
  pltpu.sync_copy(bv, acc.at[pl.ds(r0, ROWS_PER_TILE)])
  pltpu.sync_copy(bv, featsh.at[pl.ds(r0, ROWS_PER_TILE)])
  plsc.subcore_barrier()
  _edge_loop(True, p1_hbm, featsh, acc, srcv, dstv, rowsv,
             gsem, ssem_a, ssem_b)
  _drain_out(acc, av, out_hbm, c, r0)


_sc_prop1 = functools.partial(
    pl.kernel,
    out_type=jax.ShapeDtypeStruct((2, NP, D_HID), jnp.float32),
    mesh=_mesh,
    scratch_types=[
        pltpu.VMEM((NCHUNK, CHUNK), jnp.int32),
        pltpu.VMEM((NCHUNK, CHUNK), jnp.int32),
        pltpu.VMEM((16, CHUNK, D_HID), jnp.float32),
        pltpu.VMEM((ROWS_PER_TILE, D_HID), jnp.float32),
        pltpu.VMEM((ROWS_PER_TILE, D_HID), jnp.float32),
        pltpu.VMEM_SHARED((NP, D_HID), jnp.float32),
        pltpu.VMEM_SHARED((NP, D_HID), jnp.float32),
        pltpu.SemaphoreType.DMA,
        pltpu.SemaphoreType.DMA,
        pltpu.SemaphoreType.DMA,
    ],
    compiler_params=_params,
)(_prop1_body)


def _prop2_body(m1_hbm, p1_hbm, degp_hbm, b1_hbm, e_hbm, m2_hbm, hp_hbm,
                srcv, dstv, rowsv, av, bv, cv, dv, b1v,
                acc, featsh, gsem, ssem_a, ssem_b):
  c, s, wid, r0 = _tile_ids()
  _load_edges(e_hbm, 1, dstv, wid)
  _load_edges(e_hbm, 0, srcv, wid)
  pltpu.sync_copy(degp_hbm.at[0, pl.ds(r0, ROWS_PER_TILE), :], av)
  pltpu.sync_copy(degp_hbm.at[1, pl.ds(r0, ROWS_PER_TILE), :], bv)

  def dinv_loop(j4, carry):
    for k in range(4):
      j = j4 * 4 + k
      av[j] = _newton_rsqrt(av[j] + bv[j] - 1.0)
    return carry

  lax.fori_loop(0, ROWS_PER_TILE // 4, dinv_loop, 0)
  pltpu.sync_copy(p1_hbm.at[pl.ds(r0, ROWS_PER_TILE)], bv)
  pltpu.sync_copy(m1_hbm.at[0, pl.ds(r0, ROWS_PER_TILE), :], cv)
  pltpu.sync_copy(m1_hbm.at[1, pl.ds(r0, ROWS_PER_TILE), :], dv)
  pltpu.sync_copy(b1_hbm, b1v)
  b1row = b1v[0]

  def mid(j4, carry):
    for k in range(4):
      j = j4 * 4 + k
      dinv = av[j]
      m = cv[j] + dv[j] - bv[j] * dinv
      h = jnp.maximum(m * dinv + b1row, 0.0)
      dv[j] = h * dinv
    return carry

  lax.fori_loop(0, ROWS_PER_TILE // 4, mid, 0)
  pltpu.sync_copy(dv, acc.at[pl.ds(r0, ROWS_PER_TILE)])
  pltpu.sync_copy(dv, featsh.at[pl.ds(r0, ROWS_PER_TILE)])
  plsc.subcore_barrier()
  _edge_loop(True, p1_hbm, featsh, acc, srcv, dstv, rowsv,
             gsem, ssem_a, ssem_b)
  plsc.subcore_barrier()
  pltpu.sync_copy(acc.at[pl.ds(r0, ROWS_PER_TILE)], cv)

  def fin(j4, carry):
    for k in range(4):
      j = j4 * 4 + k
      cv[j] = cv[j] * av[j]
      dv[j] = dv[j] * av[j]
    return carry

  lax.fori_loop(0, ROWS_PER_TILE // 4, fin, 0)
  pltpu.sync_copy(cv, m2_hbm.at[c, pl.ds(r0, ROWS_PER_TILE), :])
  pltpu.sync_copy(dv, hp_hbm.at[pl.ds(r0, ROWS_PER_TILE)])


_sc_prop2 = functools.partial(
    pl.kernel,
    out_type=(jax.ShapeDtypeStruct((2, NP, D_HID), jnp.float32),
              jax.ShapeDtypeStruct((NP, D_HID), jnp.float32)),
    mesh=_mesh,
    scratch_types=[
        pltpu.VMEM((NCHUNK, CHUNK), jnp.int32),
        pltpu.VMEM((NCHUNK, CHUNK), jnp.int32),
        pltpu.VMEM((16, CHUNK, D_HID), jnp.float32),
        pltpu.VMEM((ROWS_PER_TILE, D_HID), jnp.float32),
        pltpu.VMEM((ROWS_PER_TILE, D_HID), jnp.float32),
        pltpu.VMEM((ROWS_PER_TILE, D_HID), jnp.float32),
        pltpu.VMEM((ROWS_PER_TILE, D_HID), jnp.float32),
        pltpu.VMEM((1, D_HID), jnp.float32),
        pltpu.VMEM_SHARED((NP, D_HID), jnp.float32),
        pltpu.VMEM_SHARED((NP, D_HID), jnp.float32),
        pltpu.SemaphoreType.DMA,
        pltpu.SemaphoreType.DMA,
        pltpu.SemaphoreType.DMA,
    ],
    compiler_params=_params,
)(_prop2_body)


def _tc_matmul(x_ref, w1_ref, out_ref):
  out_ref[...] = jnp.dot(x_ref[...], w1_ref[...],
                         preferred_element_type=jnp.float32)


def _tc_final(m_ref, hpd_ref, w2_ref, b2_ref, out_ref):
  m = (m_ref[0] + m_ref[1] - hpd_ref[...])[:N]
  out_ref[...] = jnp.dot(m, w2_ref[...],
                         preferred_element_type=jnp.float32) + b2_ref[...]


def kernel(V, E, X, W1, b1, W2, b2):
  del V
  f32 = jnp.float32
  ones = jnp.ones((NP, D_HID), f32)
  Xp = jnp.concatenate([X, jnp.zeros((NP - N, D_IN), f32)])
  E3 = E.reshape(2, ECHUNKS, CHUNK)

  degp = _sc_deg(ones, E3)
  p1 = pl.pallas_call(
      _tc_matmul,
      out_shape=jax.ShapeDtypeStruct((NP, D_HID), f32),
  )(Xp, W1)

  m1 = _sc_prop1(p1, degp, E3)
  m2, hpd = _sc_prop2(m1, p1, degp, b1.reshape(1, D_HID), E3)

  out = pl.pallas_call(
      _tc_final,
      out_shape=jax.ShapeDtypeStruct((N, D_OUT), f32),
  )(m2, hpd, W2, b2.reshape(1, D_OUT))
  return out
